# baseline (speedup 1.0000x reference)
import functools

import jax
import jax.numpy as jnp
from jax import lax
from jax.experimental import pallas as pl
from jax.experimental.pallas import tpu as pltpu
from jax.experimental.pallas import tpu_sc as plsc

NUM_ROWS = 20
EMB_DIM = 16
BATCH = 16384
PEP_LEN = 9

_info = plsc.get_sparse_core_info()
_NC, _NS, _NL = _info.num_cores, _info.num_subcores, _info.num_lanes
_NW = _NC * _NS
_B_PER_W = BATCH // _NW
_GROUPS = _B_PER_W // _NL
_LT = BATCH // 128
_LT_PER_W = _B_PER_W // 128
_TW = NUM_ROWS * EMB_DIM


def _dyn_gather(src, idx):
  return lax.gather(
      src,
      idx[:, None],
      lax.GatherDimensionNumbers(
          offset_dims=(), collapsed_slice_dims=(0,), start_index_map=(0,)
      ),
      slice_sizes=(1,),
      mode=lax.GatherScatterMode.PROMISE_IN_BOUNDS,
  )


def _make_lookup():
  mesh = plsc.VectorSubcoreMesh(core_axis_name="c", subcore_axis_name="s")

  @functools.partial(
      pl.kernel,
      mesh=mesh,
      out_type=jax.ShapeDtypeStruct((PEP_LEN, 2, _LT, 8, 128), jnp.float32),
      scratch_types=[
          pltpu.VMEM((EMB_DIM, NUM_ROWS), jnp.float32),
          pltpu.VMEM((_TW * _NL,), jnp.float32),
          pltpu.VMEM((PEP_LEN, _B_PER_W), jnp.int32),
          pltpu.VMEM((PEP_LEN, 2, _LT_PER_W, 8, 128), jnp.float32),
      ],
      compiler_params=pltpu.CompilerParams(
          use_tc_tiling_on_sc=False, needs_layout_passes=False
      ),
  )
  def lookup(table_hbm, idx_hbm, out_hbm, table_v, rep_v, idx_v, rows_v):
    wid = lax.axis_index("s") * _NC + lax.axis_index("c")
    base = wid * _B_PER_W
    pltpu.sync_copy(table_hbm, table_v)
    pltpu.sync_copy(idx_hbm.at[:, pl.ds(base, _B_PER_W)], idx_v)

    lane = lax.iota(jnp.int32, _NL)
    sel = [jnp.full((_NL,), j, jnp.int32) for j in range(_NL)]

    for d in range(EMB_DIM):
      tw0 = table_v[d, pl.ds(0, _NL)]
      tw1 = table_v[d, pl.ds(4, _NL)]
      for r in range(NUM_ROWS):
        src, j = (tw0, r) if r < _NL else (tw1, r - 4)
        val = _dyn_gather(src, sel[j])
        plsc.store_scatter(rep_v, [lane + _NL * (d * NUM_ROWS + r)], val)

    dof = [jnp.full((_NL,), _NL * NUM_ROWS * d, jnp.int32)
           for d in range(EMB_DIM)]

    def body(g, carry):
      ct = g // 8
      l0 = (g % 8) * _NL
      for p in range(PEP_LEN):
        iv = idx_v[p, pl.ds(g * _NL, _NL)]
        a = (iv << 4) + lane
        for d in range(EMB_DIM):
          rows_v[p, d // 8, ct, d % 8, pl.ds(l0, _NL)] = plsc.load_gather(
              rep_v, [a + dof[d]]
          )
      return carry

    lax.fori_loop(0, _GROUPS, body, 0)
    pltpu.sync_copy(
        rows_v, out_hbm.at[:, :, pl.ds(wid * _LT_PER_W, _LT_PER_W)]
    )

  return lookup


_lookup = _make_lookup()


@jax.jit
def kernel(x, embedding_table):
  xt = x.T.astype(jnp.int32)
  tt = embedding_table.T
  z = _lookup(tt, xt)
  out_t = z.transpose(0, 1, 3, 2, 4).reshape(PEP_LEN, EMB_DIM, BATCH)
  return out_t.transpose(2, 0, 1)


# speedup vs baseline: 1.4070x; 1.4070x over previous
import functools

import jax
import jax.numpy as jnp
from jax import lax
from jax.experimental import pallas as pl
from jax.experimental.pallas import tpu as pltpu
from jax.experimental.pallas import tpu_sc as plsc

NUM_ROWS = 20
EMB_DIM = 16
BATCH = 16384
PEP_LEN = 9

_info = plsc.get_sparse_core_info()
_NC, _NS, _NL = _info.num_cores, _info.num_subcores, _info.num_lanes
_NW = _NC * _NS
_B_PER_W = BATCH // _NW
_GROUPS = _B_PER_W // _NL
_LT = BATCH // 128
_LT_PER_W = _B_PER_W // 128


def _make_lookup():
  mesh = plsc.VectorSubcoreMesh(core_axis_name="c", subcore_axis_name="s")

  @functools.partial(
      pl.kernel,
      mesh=mesh,
      out_type=jax.ShapeDtypeStruct((PEP_LEN, 2, _LT, 8, 128), jnp.float32),
      scratch_types=[
          pltpu.VMEM((EMB_DIM, NUM_ROWS), jnp.float32),
          pltpu.VMEM((PEP_LEN, _B_PER_W), jnp.int32),
          pltpu.VMEM((PEP_LEN, 2, _LT_PER_W, 8, 128), jnp.float32),
      ],
      compiler_params=pltpu.CompilerParams(
          use_tc_tiling_on_sc=False, needs_layout_passes=False
      ),
  )
  def lookup(table_hbm, idx_hbm, out_hbm, table_v, idx_v, rows_v):
    wid = lax.axis_index("s") * _NC + lax.axis_index("c")
    base = wid * _B_PER_W
    pltpu.sync_copy(table_hbm, table_v)
    pltpu.sync_copy(idx_hbm.at[:, pl.ds(base, _B_PER_W)], idx_v)

    cols = [jnp.full((_NL,), d, jnp.int32) for d in range(EMB_DIM)]

    @plsc.parallel_loop(0, _GROUPS)
    def body(g):
      ct = g // 8
      l0 = (g % 8) * _NL
      for p in range(PEP_LEN):
        iv = idx_v[p, pl.ds(g * _NL, _NL)]
        for d in range(EMB_DIM):
          rows_v[p, d // 8, ct, d % 8, pl.ds(l0, _NL)] = plsc.load_gather(
              table_v, [cols[d], iv]
          )

    pltpu.sync_copy(
        rows_v, out_hbm.at[:, :, pl.ds(wid * _LT_PER_W, _LT_PER_W)]
    )

  return lookup


_lookup = _make_lookup()


@jax.jit
def kernel(x, embedding_table):
  xt = x.T.astype(jnp.int32)
  tt = embedding_table.T
  z = _lookup(tt, xt)
  out_t = z.transpose(0, 1, 3, 2, 4).reshape(PEP_LEN, EMB_DIM, BATCH)
  return out_t.transpose(2, 0, 1)
